# Initial kernel scaffold; baseline (speedup 1.0000x reference)
import jax
import jax.numpy as jnp
from jax.experimental import pallas as pl


def kernel(node_pos, bar_alpha, W1_0, A_0, B_0, W2_0, Ws_0, W1_1, A_1, B_1, W2_1, Ws_1, W1_2, A_2, B_2, W2_2, Ws_2, W1_3, A_3, B_3, W2_3, Ws_3):
    raise NotImplementedError("write your pallas kernel here")


# speedup vs baseline: 8.3790x; 8.3790x over previous
import numpy as np
import jax
import jax.numpy as jnp
from jax.experimental import pallas as pl

MAX_RADIUS = 6.0
NB = 10
MUL = 16
SH = 9
HID = MUL * 18

BE = 1000


def _f32dot(a, b):
    return jnp.dot(a, b, preferred_element_type=jnp.float32)


def _fused_kernel(np6_ref, ba2_ref, p1_ref, p2_ref,
                  w10, a0, b0, w20, ws0,
                  w11, a1, b1, w21, ws1,
                  w12, a2, b2, w22, ws2,
                  w13, a3, b3, w23, ws3,
                  out_ref):
    np6 = np6_ref[:]
    vx = np6[:, 1:2] - np6[:, 0:1]
    vy = np6[:, 3:4] - np6[:, 2:3]
    vz = np6[:, 5:6] - np6[:, 4:5]
    n2 = vx * vx + vy * vy + vz * vz
    n = jnp.sqrt(n2)
    inv = 1.0 / jnp.maximum(n, 1e-12)
    ux = vx * inv
    uy = vy * inv
    uz = vz * inv
    c1 = np.float32(np.sqrt(3.0))
    c2 = np.float32(np.sqrt(15.0))
    sh = jnp.concatenate([
        jnp.ones_like(ux),
        c1 * uy, c1 * uz, c1 * ux,
        c2 * ux * uy, c2 * uy * uz,
        np.float32(np.sqrt(5.0) / 2.0) * (3.0 * uz * uz - 1.0),
        c2 * ux * uz,
        np.float32(np.sqrt(15.0) / 2.0) * (ux * ux - uy * uy),
    ], axis=1)

    step = np.float32(MAX_RADIUS / (NB + 1))
    idx = jax.lax.broadcasted_iota(jnp.int32, (1, NB), 1)
    vals = (idx.astype(jnp.float32) + 1.0) * step
    diff = (n - vals) * np.float32(1.0 / step)

    def sus(t):
        safe = jnp.where(t > 0.0, t, 1.0)
        return jnp.where(t > 0.0, jnp.exp(-1.0 / safe), 0.0)

    emb = np.float32(1.14136 * np.exp(2.0) * np.sqrt(NB)) \
        * sus(diff + 1.0) * sus(1.0 - diff)

    shtile = _f32dot(sh, p2_ref[:])
    p1 = p1_ref[:]

    def edge_mult(a_ref, b_ref):
        t = jnp.tanh(_f32dot(emb, a_ref[:]))
        w = _f32dot(t, b_ref[:])
        return _f32dot(w, p1) * shtile

    m0 = edge_mult(a0, b0)
    m1 = edge_mult(a1, b1)
    m2 = edge_mult(a2, b2)
    m3 = edge_mult(a3, b3)

    ba = ba2_ref[:]
    he = ba[:, 0:1]
    ho = ba[:, 1:2]

    msg = (he * w10[:]) * m0
    ho_n = _f32dot(msg, w20[:]) + ho * ws0[:]
    he = jnp.tanh(he * ws0[:])
    ho = jnp.tanh(ho_n)

    for w1_ref, w2_ref, ws_ref, ml in ((w11, w21, ws1, m1),
                                       (w12, w22, ws2, m2)):
        msg = _f32dot(he, w1_ref[:]) * ml
        ho_n = _f32dot(msg, w2_ref[:]) + _f32dot(ho, ws_ref[:])
        he = jnp.tanh(_f32dot(he, ws_ref[:]))
        ho = jnp.tanh(ho_n)

    msg = _f32dot(he, w13[:]) * m3
    out_ref[:, 3:6] = _f32dot(msg, w23[:]) + _f32dot(ho, ws3[:])
    out_ref[:, 0:3] = _f32dot(he, ws3[:])


def kernel(node_pos, bar_alpha,
           W1_0, A_0, B_0, W2_0, Ws_0,
           W1_1, A_1, B_1, W2_1, Ws_1,
           W1_2, A_2, B_2, W2_2, Ws_2,
           W1_3, A_3, B_3, W2_3, Ws_3):
    E = node_pos.shape[0]
    np6 = node_pos.reshape(E, 6)
    ba2 = bar_alpha.reshape(E, 2)

    p1 = np.zeros((MUL, MUL * SH), np.float32)
    p2 = np.zeros((SH, MUL * SH), np.float32)
    for j in range(MUL):
        for k in range(SH):
            p1[j, SH * j + k] = 1.0
            p2[k, SH * j + k] = 1.0
    p1 = jnp.asarray(p1)
    p2 = jnp.asarray(p2)

    weights = (W1_0, A_0, B_0, W2_0, Ws_0,
               W1_1, A_1, B_1, W2_1, Ws_1,
               W1_2, A_2, B_2, W2_2, Ws_2,
               W1_3, A_3, B_3, W2_3, Ws_3)

    def rep_spec(w):
        return pl.BlockSpec(w.shape, lambda i: (0,) * w.ndim)

    grid = E // BE
    out = pl.pallas_call(
        _fused_kernel,
        grid=(grid,),
        in_specs=[
            pl.BlockSpec((BE, 6), lambda i: (i, 0)),
            pl.BlockSpec((BE, 2), lambda i: (i, 0)),
            rep_spec(p1),
            rep_spec(p2),
        ] + [rep_spec(w) for w in weights],
        out_specs=pl.BlockSpec((BE, 6), lambda i: (i, 0)),
        out_shape=jax.ShapeDtypeStruct((E, 6), jnp.float32),
    )(np6, ba2, p1, p2, *weights)

    return out.reshape(2 * E, 3)


# speedup vs baseline: 9.1901x; 1.0968x over previous
import numpy as np
import jax
import jax.numpy as jnp
from jax.experimental import pallas as pl

MAX_RADIUS = 6.0
NB = 10
MUL = 16
SH = 9
HID = MUL * 18

BE = 2000


def _f32dot(a, b):
    return jnp.dot(a, b, preferred_element_type=jnp.float32)


def _fused_kernel(np6_ref, ba2_ref, aall_ref, bbd_ref, p2_ref,
                  p1_0, p1_1, p1_2, p1_3,
                  w10, w20, ws0,
                  w11, w21, ws1,
                  w12, w22, ws2,
                  w13, w23, ws3,
                  out_ref):
    np6 = np6_ref[:]
    vx = np6[:, 1:2] - np6[:, 0:1]
    vy = np6[:, 3:4] - np6[:, 2:3]
    vz = np6[:, 5:6] - np6[:, 4:5]
    n2 = vx * vx + vy * vy + vz * vz
    n = jnp.sqrt(n2)
    inv = 1.0 / jnp.maximum(n, 1e-12)
    ux = vx * inv
    uy = vy * inv
    uz = vz * inv
    c1 = np.float32(np.sqrt(3.0))
    c2 = np.float32(np.sqrt(15.0))
    sh = jnp.concatenate([
        jnp.ones_like(ux),
        c1 * uy, c1 * uz, c1 * ux,
        c2 * ux * uy, c2 * uy * uz,
        np.float32(np.sqrt(5.0) / 2.0) * (3.0 * uz * uz - 1.0),
        c2 * ux * uz,
        np.float32(np.sqrt(15.0) / 2.0) * (ux * ux - uy * uy),
    ], axis=1)

    step = np.float32(MAX_RADIUS / (NB + 1))
    idx = jax.lax.broadcasted_iota(jnp.int32, (1, NB), 1)
    vals = (idx.astype(jnp.float32) + 1.0) * step
    diff = (n - vals) * np.float32(1.0 / step)

    def sus(t):
        safe = jnp.where(t > 0.0, t, 1.0)
        return jnp.where(t > 0.0, jnp.exp(-1.0 / safe), 0.0)

    emb = np.float32(1.14136 * np.exp(2.0) * np.sqrt(NB)) \
        * sus(diff + 1.0) * sus(1.0 - diff)

    t = jnp.tanh(_f32dot(emb, aall_ref[:]))
    w_all = _f32dot(t, bbd_ref[:])
    shtile = _f32dot(sh, p2_ref[:])

    def mult(p1_ref):
        return _f32dot(w_all, p1_ref[:]) * shtile

    m0 = mult(p1_0)
    m1 = mult(p1_1)
    m2 = mult(p1_2)
    m3 = mult(p1_3)

    ba = ba2_ref[:]
    he = ba[:, 0:1]
    ho = ba[:, 1:2]

    msg = (he * w10[:]) * m0
    u = _f32dot(msg, w20[:])
    hE = jnp.tanh(he * ws0[:])
    hO = jnp.tanh(ho * ws0[:] + u)
    h = jnp.concatenate([hE, hO], axis=0)

    for w1_ref, w2_ref, ws_ref, ml in ((w11, w21, ws1, m1),
                                       (w12, w22, ws2, m2)):
        hs = _f32dot(h, ws_ref[:])
        msg = _f32dot(h[:BE], w1_ref[:]) * ml
        u = _f32dot(msg, w2_ref[:])
        h = jnp.tanh(jnp.concatenate([hs[:BE], hs[BE:] + u], axis=0))

    hs = _f32dot(h, ws3[:])
    msg = _f32dot(h[:BE], w13[:]) * m3
    out_ref[:, 0:3] = hs[:BE]
    out_ref[:, 3:6] = hs[BE:] + _f32dot(msg, w23[:])


def kernel(node_pos, bar_alpha,
           W1_0, A_0, B_0, W2_0, Ws_0,
           W1_1, A_1, B_1, W2_1, Ws_1,
           W1_2, A_2, B_2, W2_2, Ws_2,
           W1_3, A_3, B_3, W2_3, Ws_3):
    E = node_pos.shape[0]
    np6 = node_pos.reshape(E, 6)
    ba2 = bar_alpha.reshape(E, 2)

    a_all = jnp.concatenate([A_0, A_1, A_2, A_3], axis=1)
    b_bd = jnp.zeros((4 * 100, 4 * MUL), jnp.float32)
    for l, B in enumerate((B_0, B_1, B_2, B_3)):
        b_bd = b_bd.at[100 * l:100 * (l + 1),
                       MUL * l:MUL * (l + 1)].set(B)

    p2 = np.zeros((SH, MUL * SH), np.float32)
    p1s = []
    for l in range(4):
        p1 = np.zeros((4 * MUL, MUL * SH), np.float32)
        for j in range(MUL):
            for k in range(SH):
                p1[MUL * l + j, SH * j + k] = 1.0
                p2[k, SH * j + k] = 1.0
        p1s.append(jnp.asarray(p1))
    p2 = jnp.asarray(p2)

    ops = [np6, ba2, a_all, b_bd, p2] + p1s + [
        W1_0, W2_0, Ws_0,
        W1_1, W2_1, Ws_1,
        W1_2, W2_2, Ws_2,
        W1_3, W2_3, Ws_3,
    ]

    def rep_spec(w):
        return pl.BlockSpec(w.shape, lambda i: (0,) * w.ndim)

    grid = E // BE
    out = pl.pallas_call(
        _fused_kernel,
        grid=(grid,),
        in_specs=[
            pl.BlockSpec((BE, 6), lambda i: (i, 0)),
            pl.BlockSpec((BE, 2), lambda i: (i, 0)),
        ] + [rep_spec(w) for w in ops[2:]],
        out_specs=pl.BlockSpec((BE, 6), lambda i: (i, 0)),
        out_shape=jax.ShapeDtypeStruct((E, 6), jnp.float32),
    )(*ops)

    return out.reshape(2 * E, 3)


# speedup vs baseline: 12.2842x; 1.3367x over previous
import numpy as np
import jax
import jax.numpy as jnp
from jax.experimental import pallas as pl

MAX_RADIUS = 6.0
NB = 10
MUL = 16
SH = 9
HID = MUL * 18

BE = 2048
EPAD = 25 * BE


def _f32dot(a, b):
    return jnp.dot(a, b, preferred_element_type=jnp.float32)


def _fused_kernel(np6_ref, ba2_ref, aall_ref, bbd_ref, p2_ref,
                  p1_0, p1_1, p1_2, p1_3,
                  w10, w20, ws0,
                  w11, w21, ws1,
                  w12, w22, ws2,
                  w13, w23, ws3,
                  out_ref):
    np6 = np6_ref[:]
    vx = np6[1:2, :] - np6[0:1, :]
    vy = np6[3:4, :] - np6[2:3, :]
    vz = np6[5:6, :] - np6[4:5, :]
    n2 = vx * vx + vy * vy + vz * vz
    n = jnp.sqrt(n2)
    inv = 1.0 / jnp.maximum(n, 1e-12)
    ux = vx * inv
    uy = vy * inv
    uz = vz * inv
    c1 = np.float32(np.sqrt(3.0))
    c2 = np.float32(np.sqrt(15.0))
    sh = jnp.concatenate([
        jnp.ones_like(ux),
        c1 * uy, c1 * uz, c1 * ux,
        c2 * ux * uy, c2 * uy * uz,
        np.float32(np.sqrt(5.0) / 2.0) * (3.0 * uz * uz - 1.0),
        c2 * ux * uz,
        np.float32(np.sqrt(15.0) / 2.0) * (ux * ux - uy * uy),
    ], axis=0)

    step = np.float32(MAX_RADIUS / (NB + 1))
    idx = jax.lax.broadcasted_iota(jnp.int32, (NB, 1), 0)
    vals = (idx.astype(jnp.float32) + 1.0) * step
    diff = (n - vals) * np.float32(1.0 / step)

    def sus(t):
        safe = jnp.where(t > 0.0, t, 1.0)
        return jnp.where(t > 0.0, jnp.exp(-1.0 / safe), 0.0)

    emb = np.float32(1.14136 * np.exp(2.0) * np.sqrt(NB)) \
        * sus(diff + 1.0) * sus(1.0 - diff)

    t = jnp.tanh(_f32dot(aall_ref[:], emb))
    w_all = _f32dot(bbd_ref[:], t)
    shtile = _f32dot(p2_ref[:], sh)

    def mult(p1_ref):
        return _f32dot(p1_ref[:], w_all) * shtile

    m0 = mult(p1_0)
    m1 = mult(p1_1)
    m2 = mult(p1_2)
    m3 = mult(p1_3)

    ba = ba2_ref[:]
    he = ba[0:1, :]
    ho = ba[1:2, :]

    msg = (w10[:] * he) * m0
    u = _f32dot(w20[:], msg)
    hE = jnp.tanh(ws0[:] * he)
    hO = jnp.tanh(ws0[:] * ho + u)

    for w1_ref, w2_ref, ws_ref, ml in ((w11, w21, ws1, m1),
                                       (w12, w22, ws2, m2)):
        msg = _f32dot(w1_ref[:], hE) * ml
        u = _f32dot(w2_ref[:], msg)
        hO = jnp.tanh(_f32dot(ws_ref[:], hO) + u)
        hE = jnp.tanh(_f32dot(ws_ref[:], hE))

    msg = _f32dot(w13[:], hE) * m3
    out_ref[0:3, :] = _f32dot(ws3[:], hE)
    out_ref[3:6, :] = _f32dot(ws3[:], hO) + _f32dot(w23[:], msg)


def kernel(node_pos, bar_alpha,
           W1_0, A_0, B_0, W2_0, Ws_0,
           W1_1, A_1, B_1, W2_1, Ws_1,
           W1_2, A_2, B_2, W2_2, Ws_2,
           W1_3, A_3, B_3, W2_3, Ws_3):
    E = node_pos.shape[0]
    pad = EPAD - E
    np6 = jnp.pad(node_pos.reshape(E, 6).T, ((0, 0), (0, pad)))
    ba2 = jnp.pad(bar_alpha.reshape(E, 2).T, ((0, 0), (0, pad)))

    a_all = jnp.concatenate([A_0, A_1, A_2, A_3], axis=1).T
    b_bd = jnp.zeros((4 * 100, 4 * MUL), jnp.float32)
    for l, B in enumerate((B_0, B_1, B_2, B_3)):
        b_bd = b_bd.at[100 * l:100 * (l + 1),
                       MUL * l:MUL * (l + 1)].set(B)
    b_bd = b_bd.T

    p2 = np.zeros((MUL * SH, SH), np.float32)
    p1s = []
    for l in range(4):
        p1 = np.zeros((MUL * SH, 4 * MUL), np.float32)
        for j in range(MUL):
            for k in range(SH):
                p1[SH * j + k, MUL * l + j] = 1.0
                p2[SH * j + k, k] = 1.0
        p1s.append(jnp.asarray(p1))
    p2 = jnp.asarray(p2)

    ops = [np6, ba2, a_all, b_bd, p2] + p1s + [
        W1_0.T, W2_0.T, Ws_0.T,
        W1_1.T, W2_1.T, Ws_1.T,
        W1_2.T, W2_2.T, Ws_2.T,
        W1_3.T, W2_3.T, Ws_3.T,
    ]

    def rep_spec(w):
        return pl.BlockSpec(w.shape, lambda i: (0,) * w.ndim)

    grid = EPAD // BE
    out = pl.pallas_call(
        _fused_kernel,
        grid=(grid,),
        in_specs=[
            pl.BlockSpec((6, BE), lambda i: (0, i)),
            pl.BlockSpec((2, BE), lambda i: (0, i)),
        ] + [rep_spec(w) for w in ops[2:]],
        out_specs=pl.BlockSpec((6, BE), lambda i: (0, i)),
        out_shape=jax.ShapeDtypeStruct((6, EPAD), jnp.float32),
    )(*ops)

    return out[:, :E].reshape(2, 3, E).transpose(2, 0, 1).reshape(2 * E, 3)


# speedup vs baseline: 12.3364x; 1.0043x over previous
import numpy as np
import jax
import jax.numpy as jnp
from jax.experimental import pallas as pl

MAX_RADIUS = 6.0
NB = 10
MUL = 16
SH = 9
HID = MUL * 18

BE = 2048
EPAD = 25 * BE


def _f32dot(a, b):
    return jnp.dot(a, b, preferred_element_type=jnp.float32)


def _fused_kernel(np6_ref, ba2_ref, aall_ref, bbd_ref, p2_ref,
                  p1_0, p1_1, p1_2, p1_3,
                  w10, w20, ws0,
                  w11, w21, ws1,
                  w12, w22, ws2,
                  w13, w23, ws3,
                  out_ref):
    np6 = np6_ref[:]
    vx = np6[1:2, :] - np6[0:1, :]
    vy = np6[3:4, :] - np6[2:3, :]
    vz = np6[5:6, :] - np6[4:5, :]
    n2 = vx * vx + vy * vy + vz * vz
    n = jnp.sqrt(n2)
    inv = 1.0 / jnp.maximum(n, 1e-12)
    ux = vx * inv
    uy = vy * inv
    uz = vz * inv
    c1 = np.float32(np.sqrt(3.0))
    c2 = np.float32(np.sqrt(15.0))
    sh = jnp.concatenate([
        jnp.ones_like(ux),
        c1 * uy, c1 * uz, c1 * ux,
        c2 * ux * uy, c2 * uy * uz,
        np.float32(np.sqrt(5.0) / 2.0) * (3.0 * uz * uz - 1.0),
        c2 * ux * uz,
        np.float32(np.sqrt(15.0) / 2.0) * (ux * ux - uy * uy),
    ], axis=0)

    step = np.float32(MAX_RADIUS / (NB + 1))
    idx = jax.lax.broadcasted_iota(jnp.int32, (NB, 1), 0)
    vals = (idx.astype(jnp.float32) + 1.0) * step
    diff = (n - vals) * np.float32(1.0 / step)

    def sus(t):
        safe = jnp.where(t > 0.0, t, 1.0)
        return jnp.where(t > 0.0, jnp.exp(-1.0 / safe), 0.0)

    emb = np.float32(1.14136 * np.exp(2.0) * np.sqrt(NB)) \
        * sus(diff + 1.0) * sus(1.0 - diff)

    t = jnp.tanh(_f32dot(aall_ref[:], emb))
    w_all = _f32dot(bbd_ref[:], t)
    shtile = _f32dot(p2_ref[:], sh)

    def mult(p1_ref):
        return _f32dot(p1_ref[:], w_all) * shtile

    m0 = mult(p1_0)
    m1 = mult(p1_1)
    m2 = mult(p1_2)
    m3 = mult(p1_3)

    ba = ba2_ref[:]
    h0 = jnp.concatenate([ba[0:1, :], ba[1:2, :]], axis=1)

    msg = (w10[:] * h0[:, :BE]) * m0
    u = _f32dot(w20[:], msg)
    y = ws0[:] * h0
    h = jnp.tanh(jnp.concatenate([y[:, :BE], y[:, BE:] + u], axis=1))

    for w1_ref, w2_ref, ws_ref, ml in ((w11, w21, ws1, m1),
                                       (w12, w22, ws2, m2)):
        y = _f32dot(ws_ref[:], h)
        msg = _f32dot(w1_ref[:], h[:, :BE]) * ml
        u = _f32dot(w2_ref[:], msg)
        h = jnp.tanh(jnp.concatenate([y[:, :BE], y[:, BE:] + u], axis=1))

    y = _f32dot(ws3[:], h)
    msg = _f32dot(w13[:], h[:, :BE]) * m3
    out_ref[0:3, :] = y[:, :BE]
    out_ref[3:6, :] = y[:, BE:] + _f32dot(w23[:], msg)


def kernel(node_pos, bar_alpha,
           W1_0, A_0, B_0, W2_0, Ws_0,
           W1_1, A_1, B_1, W2_1, Ws_1,
           W1_2, A_2, B_2, W2_2, Ws_2,
           W1_3, A_3, B_3, W2_3, Ws_3):
    E = node_pos.shape[0]
    pad = EPAD - E
    np6 = jnp.pad(node_pos.reshape(E, 6).T, ((0, 0), (0, pad)))
    ba2 = jnp.pad(bar_alpha.reshape(E, 2).T, ((0, 0), (0, pad)))

    a_all = jnp.concatenate([A_0, A_1, A_2, A_3], axis=1).T
    b_bd = jnp.zeros((4 * 100, 4 * MUL), jnp.float32)
    for l, B in enumerate((B_0, B_1, B_2, B_3)):
        b_bd = b_bd.at[100 * l:100 * (l + 1),
                       MUL * l:MUL * (l + 1)].set(B)
    b_bd = b_bd.T

    p2 = np.zeros((MUL * SH, SH), np.float32)
    p1s = []
    for l in range(4):
        p1 = np.zeros((MUL * SH, 4 * MUL), np.float32)
        for j in range(MUL):
            for k in range(SH):
                p1[SH * j + k, MUL * l + j] = 1.0
                p2[SH * j + k, k] = 1.0
        p1s.append(jnp.asarray(p1))
    p2 = jnp.asarray(p2)

    ops = [np6, ba2, a_all, b_bd, p2] + p1s + [
        W1_0.T, W2_0.T, Ws_0.T,
        W1_1.T, W2_1.T, Ws_1.T,
        W1_2.T, W2_2.T, Ws_2.T,
        W1_3.T, W2_3.T, Ws_3.T,
    ]

    def rep_spec(w):
        return pl.BlockSpec(w.shape, lambda i: (0,) * w.ndim)

    grid = EPAD // BE
    out = pl.pallas_call(
        _fused_kernel,
        grid=(grid,),
        in_specs=[
            pl.BlockSpec((6, BE), lambda i: (0, i)),
            pl.BlockSpec((2, BE), lambda i: (0, i)),
        ] + [rep_spec(w) for w in ops[2:]],
        out_specs=pl.BlockSpec((6, BE), lambda i: (0, i)),
        out_shape=jax.ShapeDtypeStruct((6, EPAD), jnp.float32),
    )(*ops)

    return out[:, :E].reshape(2, 3, E).transpose(2, 0, 1).reshape(2 * E, 3)


# speedup vs baseline: 15.3253x; 1.2423x over previous
import numpy as np
import jax
import jax.numpy as jnp
from jax.experimental import pallas as pl

MAX_RADIUS = 6.0
NB = 10
MUL = 16
SH = 9
HID = MUL * 18

BE = 2048
EPAD = 25 * BE


def _f32dot(a, b):
    return jnp.dot(a, b, preferred_element_type=jnp.float32)


def _fused_kernel(np6_ref, ba2_ref, p2_ref, p1_ref,
                  a0, b0, a1, b1, a2, b2, a3, b3,
                  w10, w20, ws0,
                  w11, w21, ws1,
                  w12, w22, ws2,
                  w13, w23, ws3,
                  out_ref):
    np6 = np6_ref[:]
    vx = np6[1:2, :] - np6[0:1, :]
    vy = np6[3:4, :] - np6[2:3, :]
    vz = np6[5:6, :] - np6[4:5, :]
    n2 = vx * vx + vy * vy + vz * vz
    n = jnp.sqrt(n2)
    inv = 1.0 / jnp.maximum(n, 1e-12)
    ux = vx * inv
    uy = vy * inv
    uz = vz * inv
    c1 = np.float32(np.sqrt(3.0))
    c2 = np.float32(np.sqrt(15.0))
    sh = jnp.concatenate([
        jnp.ones_like(ux),
        c1 * uy, c1 * uz, c1 * ux,
        c2 * ux * uy, c2 * uy * uz,
        np.float32(np.sqrt(5.0) / 2.0) * (3.0 * uz * uz - 1.0),
        c2 * ux * uz,
        np.float32(np.sqrt(15.0) / 2.0) * (ux * ux - uy * uy),
    ], axis=0)

    step = np.float32(MAX_RADIUS / (NB + 1))
    idx = jax.lax.broadcasted_iota(jnp.int32, (NB, 1), 0)
    vals = (idx.astype(jnp.float32) + 1.0) * step
    diff = (n - vals) * np.float32(1.0 / step)

    def sus(t):
        safe = jnp.where(t > 0.0, t, 1.0)
        return jnp.where(t > 0.0, jnp.exp(-1.0 / safe), 0.0)

    emb = np.float32(1.14136 * np.exp(2.0) * np.sqrt(NB)) \
        * sus(diff + 1.0) * sus(1.0 - diff)

    shtile = _f32dot(p2_ref[:], sh)
    p1c = p1_ref[:]

    def mult(a_ref, b_ref):
        t = jnp.tanh(_f32dot(a_ref[:], emb))
        w = _f32dot(b_ref[:], t)
        return _f32dot(p1c, w) * shtile

    m0 = mult(a0, b0)
    m1 = mult(a1, b1)
    m2 = mult(a2, b2)
    m3 = mult(a3, b3)

    ba = ba2_ref[:]
    h0 = jnp.concatenate([ba[0:1, :], ba[1:2, :]], axis=1)

    msg = (w10[:] * h0[:, :BE]) * m0
    u = _f32dot(w20[:], msg)
    y = ws0[:] * h0
    h = jnp.tanh(jnp.concatenate([y[:, :BE], y[:, BE:] + u], axis=1))

    for w1_ref, w2_ref, ws_ref, ml in ((w11, w21, ws1, m1),
                                       (w12, w22, ws2, m2)):
        y = _f32dot(ws_ref[:], h)
        msg = _f32dot(w1_ref[:], h[:, :BE]) * ml
        u = _f32dot(w2_ref[:], msg)
        h = jnp.tanh(jnp.concatenate([y[:, :BE], y[:, BE:] + u], axis=1))

    y = _f32dot(ws3[:], h)
    msg = _f32dot(w13[:], h[:, :BE]) * m3
    out_ref[0:3, :] = y[:, :BE]
    out_ref[3:6, :] = y[:, BE:] + _f32dot(w23[:], msg)


def kernel(node_pos, bar_alpha,
           W1_0, A_0, B_0, W2_0, Ws_0,
           W1_1, A_1, B_1, W2_1, Ws_1,
           W1_2, A_2, B_2, W2_2, Ws_2,
           W1_3, A_3, B_3, W2_3, Ws_3):
    E = node_pos.shape[0]
    pad = EPAD - E
    np6 = jnp.pad(node_pos.reshape(E, 6).T, ((0, 0), (0, pad)))
    ba2 = jnp.pad(jnp.stack([bar_alpha[0::2], bar_alpha[1::2]]),
                  ((0, 0), (0, pad)))

    p2 = np.zeros((MUL * SH, SH), np.float32)
    p1c = np.zeros((MUL * SH, MUL), np.float32)
    for j in range(MUL):
        for k in range(SH):
            p1c[SH * j + k, j] = 1.0
            p2[SH * j + k, k] = 1.0
    p1c = jnp.asarray(p1c)
    p2 = jnp.asarray(p2)

    ops = [np6, ba2, p2, p1c,
           A_0.T, B_0.T, A_1.T, B_1.T, A_2.T, B_2.T, A_3.T, B_3.T,
           W1_0.T, W2_0.T, Ws_0.T,
           W1_1.T, W2_1.T, Ws_1.T,
           W1_2.T, W2_2.T, Ws_2.T,
           W1_3.T, W2_3.T, Ws_3.T]

    def rep_spec(w):
        return pl.BlockSpec(w.shape, lambda i: (0,) * w.ndim)

    grid = EPAD // BE
    out = pl.pallas_call(
        _fused_kernel,
        grid=(grid,),
        in_specs=[
            pl.BlockSpec((6, BE), lambda i: (0, i)),
            pl.BlockSpec((2, BE), lambda i: (0, i)),
        ] + [rep_spec(w) for w in ops[2:]],
        out_specs=pl.BlockSpec((6, BE), lambda i: (0, i)),
        out_shape=jax.ShapeDtypeStruct((6, EPAD), jnp.float32),
    )(*ops)

    return jnp.swapaxes(out, 0, 1)[:E].reshape(2 * E, 3)
